# baseline (speedup 1.0000x reference)
import functools

import jax
import jax.numpy as jnp
from jax import lax
from jax.experimental import pallas as pl
from jax.experimental.pallas import tpu as pltpu
from jax.experimental.pallas import tpu_sc as plsc

N = 20000
C = 64
MAX_OUT = 1000
IOU_THRESH = 0.5
BLK = 128
NB = (N + BLK - 1) // BLK
NPAD = NB * BLK
KBUF = MAX_OUT + BLK + 24
NKCH = 3
KCH = KBUF // NKCH
GPAD = 1024
_NC, _NS = 2, 16
_NW = _NC * _NS
_RPW = GPAD // _NW
NBP = 160
_BPW = NBP // _NW


def _nms_scan_body(sboxes_ref, ord_ref, sc_ref, pad_ref,
                   out_ref, obox_ref, oscr_ref, kept_ref, kmeta_ref):
    lane_i = lax.broadcasted_iota(jnp.int32, (1, BLK), 1)
    sub_b = lax.broadcasted_iota(jnp.int32, (BLK, 1), 0).astype(jnp.float32)
    row_c = lax.broadcasted_iota(jnp.int32, (KCH, 1), 0)
    eye = (lax.broadcasted_iota(jnp.int32, (BLK, BLK), 0)
           == lax.broadcasted_iota(jnp.int32, (BLK, BLK), 1)).astype(jnp.float32)
    lt = (lax.broadcasted_iota(jnp.int32, (BLK, BLK), 0)
          <= lax.broadcasted_iota(jnp.int32, (BLK, BLK), 1)).astype(jnp.float32)

    def exact_mm(a, b, dims):
        return lax.dot_general(a, b, (dims, ((), ())),
                               precision=lax.Precision.HIGHEST,
                               preferred_element_type=jnp.float32)

    def body(state):
        b, cnt = state
        blk = sboxes_ref[pl.ds(b, 1)].reshape(4, BLK)
        ord_row = ord_ref[pl.ds(b, 1)].reshape(1, BLK)
        sc_row = sc_ref[pl.ds(b, 1)].reshape(1, BLK)
        cx1, cy1 = blk[0:1, :], blk[1:2, :]
        cx2, cy2 = blk[2:3, :], blk[3:4, :]
        area_c = (cx2 - cx1) * (cy2 - cy1)

        def kept_chunk(c):
            lo = c * KCH
            kx1 = kept_ref[0, lo:lo + KCH, :]
            ky1 = kept_ref[1, lo:lo + KCH, :]
            kx2 = kept_ref[2, lo:lo + KCH, :]
            ky2 = kept_ref[3, lo:lo + KCH, :]
            area_k = kept_ref[4, lo:lo + KCH, :]
            ix1 = jnp.maximum(kx1, cx1)
            iy1 = jnp.maximum(ky1, cy1)
            ix2 = jnp.minimum(kx2, cx2)
            iy2 = jnp.minimum(ky2, cy2)
            inter = jnp.maximum(ix2 - ix1, 0.0) * jnp.maximum(iy2 - iy1, 0.0)
            iou = inter / ((area_k + area_c) - inter + 1e-9)
            supp = (iou > IOU_THRESH) & ((row_c + lo) < cnt)
            return jnp.max(supp.astype(jnp.float32), axis=0, keepdims=True)

        supp_any = kept_chunk(0)
        for c in range(1, NKCH):
            m = lax.cond(cnt > c * KCH, lambda c=c: kept_chunk(c),
                         lambda: jnp.zeros((1, BLK), jnp.float32))
            supp_any = jnp.maximum(supp_any, m)
        gidx = b * BLK + lane_i
        alive = (supp_any < 0.5) & (gidx < N)

        candT = exact_mm(eye, blk, ((1,), (1,)))
        ax1, ay1 = candT[:, 0:1], candT[:, 1:2]
        ax2, ay2 = candT[:, 2:3], candT[:, 3:4]
        area_a = (ax2 - ax1) * (ay2 - ay1)
        jx1 = jnp.maximum(ax1, cx1)
        jy1 = jnp.maximum(ay1, cy1)
        jx2 = jnp.minimum(ax2, cx2)
        jy2 = jnp.minimum(ay2, cy2)
        jinter = jnp.maximum(jx2 - jx1, 0.0) * jnp.maximum(jy2 - jy1, 0.0)
        jiou = jinter / ((area_a + area_c) - jinter + 1e-9)
        sub_i2 = lax.broadcasted_iota(jnp.int32, (BLK, BLK), 0)
        lane_i2 = lax.broadcasted_iota(jnp.int32, (BLK, BLK), 1)
        sup_f = ((jiou > IOU_THRESH) & (lane_i2 > sub_i2)).astype(jnp.float32)

        base = alive

        def fix_body(st):
            a_f, _ = st
            killed = lax.dot_general(a_f, sup_f,
                                     (((1,), (0,)), ((), ())),
                                     preferred_element_type=jnp.float32)
            a_new = (base & (killed < 0.5)).astype(jnp.float32)
            return a_new, jnp.any(a_new != a_f)

        alive_f0 = base.astype(jnp.float32)
        alive_fix, _ = lax.while_loop(lambda st: st[1], fix_body,
                                      (alive_f0, jnp.bool_(True)))
        alive = alive_fix > 0.5

        alive_f = alive.astype(jnp.float32)
        cum = exact_mm(alive_f, lt, ((1,), (0,)))
        n_alive = jnp.sum(alive_f).astype(jnp.int32)
        onehot = ((sub_b == (cum - 1.0)) & alive).astype(jnp.float32)
        geom = jnp.concatenate([candT, area_a], axis=1)
        geom_new = exact_mm(onehot, geom, ((1,), (0,)))
        for c in range(5):
            kept_ref[c, pl.ds(cnt, BLK), :] = jnp.broadcast_to(
                geom_new[:, c:c + 1], (BLK, BLK))
        rows = jnp.concatenate([ord_row, sc_row], axis=0)
        rows_new = exact_mm(onehot, rows, ((1,), (1,)))
        meta = jnp.concatenate(
            [geom_new[:, 0:4], rows_new,
             jnp.zeros((BLK, 2), jnp.float32)], axis=1)
        kmeta_ref[pl.ds(cnt, BLK), :] = meta
        return b + 1, cnt + n_alive

    def cond(state):
        b, cnt = state
        return (b < NB) & (cnt < MAX_OUT)

    _, cnt_fin = lax.while_loop(cond, body, (jnp.int32(0), jnp.int32(0)))

    row_g = lax.broadcasted_iota(jnp.int32, (GPAD, 1), 0)
    sel = kmeta_ref[0:GPAD, 4:5]
    out_ref[:, :] = jnp.where((row_g < cnt_fin) & (row_g < MAX_OUT),
                              sel, 0.0).astype(jnp.int32)
    row_o = lax.broadcasted_iota(jnp.int32, (MAX_OUT, 1), 0)
    live = row_o < cnt_fin
    obox_ref[:, :] = jnp.where(live, kmeta_ref[0:MAX_OUT, 0:4],
                               pad_ref[0:1, 0:4])
    oscr_ref[:, :] = jnp.where(live, kmeta_ref[0:MAX_OUT, 5:6],
                               pad_ref[0:1, 4:5])


def _nms_scan(sboxes_blk, ord_blk, sc_blk, pad_row):
    return pl.pallas_call(
        _nms_scan_body,
        out_shape=(jax.ShapeDtypeStruct((GPAD, 1), jnp.int32),
                   jax.ShapeDtypeStruct((MAX_OUT, 4), jnp.float32),
                   jax.ShapeDtypeStruct((MAX_OUT, 1), jnp.float32)),
        scratch_shapes=[
            pltpu.VMEM((5, KBUF, BLK), jnp.float32),
            pltpu.VMEM((KBUF, 8), jnp.float32),
        ],
    )(sboxes_blk, ord_blk, sc_blk, pad_row)


def _sc_layout_body(bflat_hbm, ordp_hbm, out_hbm, idx_v, il_v, tbuf_v, sem):
    wid = lax.axis_index("s") * _NC + lax.axis_index("c")
    base = wid * _BPW * BLK
    pltpu.sync_copy(ordp_hbm.at[pl.ds(base, _BPW * BLK)], idx_v)
    for gg in range(_BPW):
        for c in range(4):
            for h in range(BLK // 16):
                part = idx_v[pl.ds(gg * BLK + h * 16, 16)] * 4 + c
                il_v[gg, c, pl.ds(h * 16, 16)] = part
    copies = [pltpu.async_copy(bflat_hbm.at[il_v.at[gg, c]],
                               tbuf_v.at[gg, c], sem)
              for gg in range(_BPW) for c in range(4)]
    for cp in copies:
        cp.wait()
    pltpu.sync_copy(tbuf_v, out_hbm.at[pl.ds(wid * _BPW, _BPW)])


@functools.lru_cache(maxsize=1)
def _sc_layout():
    return pl.kernel(
        _sc_layout_body,
        out_type=jax.ShapeDtypeStruct((NBP, 4, BLK), jnp.float32),
        mesh=plsc.VectorSubcoreMesh(core_axis_name="c", subcore_axis_name="s"),
        scratch_types=[
            pltpu.VMEM((_BPW * BLK,), jnp.int32),
            pltpu.VMEM((_BPW, 4, BLK), jnp.int32),
            pltpu.VMEM((_BPW, 4, BLK), jnp.float32),
            pltpu.SemaphoreType.DMA,
        ],
        compiler_params=pltpu.CompilerParams(use_tc_tiling_on_sc=False),
    )


def _sc_gather_body(feats_hbm, sel_hbm, out_hbm, idx_v, buf, sem):
    wid = lax.axis_index("s") * _NC + lax.axis_index("c")
    base = wid * _RPW
    pltpu.sync_copy(sel_hbm.at[pl.ds(base, _RPW)], idx_v)
    pltpu.async_copy(feats_hbm.at[idx_v], buf, sem).wait()
    pltpu.sync_copy(buf, out_hbm.at[pl.ds(base, _RPW)])


@functools.lru_cache(maxsize=1)
def _sc_gather():
    return pl.kernel(
        _sc_gather_body,
        out_type=jax.ShapeDtypeStruct((GPAD, C), jnp.float32),
        mesh=plsc.VectorSubcoreMesh(core_axis_name="c", subcore_axis_name="s"),
        scratch_types=[
            pltpu.VMEM((_RPW,), jnp.int32),
            pltpu.VMEM((_RPW, C), jnp.float32),
            pltpu.SemaphoreType.DMA,
        ],
        compiler_params=pltpu.CompilerParams(use_tc_tiling_on_sc=False),
    )


def kernel(boxes, scores, feats):
    iota = lax.iota(jnp.int32, N)
    neg_sorted, order = lax.sort((-scores, iota), num_keys=1)
    order_pad = jnp.pad(order, (0, NBP * BLK - N))
    sboxes_blk = _sc_layout()(boxes.reshape(-1), order_pad)
    ord_blk = jnp.pad(order.astype(jnp.float32),
                      (0, NPAD - N)).reshape(NB, 1, BLK)
    sc_blk = jnp.pad(-neg_sorted, (0, NPAD - N)).reshape(NB, 1, BLK)
    pad_row = jnp.concatenate(
        [boxes[0:1, :], scores[0:1, None], jnp.zeros((1, 3), jnp.float32)],
        axis=1)

    sel, kept_boxes, kept_scores = _nms_scan(sboxes_blk, ord_blk, sc_blk,
                                             pad_row)

    feats_g = _sc_gather()(feats, sel.reshape(-1))

    return kept_boxes, kept_scores.reshape(-1), feats_g[:MAX_OUT, :]


# speedup vs baseline: 1.0075x; 1.0075x over previous
import functools

import jax
import jax.numpy as jnp
from jax import lax
from jax.experimental import pallas as pl
from jax.experimental.pallas import tpu as pltpu
from jax.experimental.pallas import tpu_sc as plsc

N = 20000
C = 64
MAX_OUT = 1000
IOU_THRESH = 0.5
BLK = 128
NB = (N + BLK - 1) // BLK
NPAD = NB * BLK
KBUF = MAX_OUT + BLK + 24
NKCH = 3
KCH = KBUF // NKCH
GPAD = 1024
_NC, _NS = 2, 16
_NW = _NC * _NS
_RPW = GPAD // _NW
NBP = 160
_BPW = NBP // _NW
_GRW = 40
_GW = MAX_OUT // _GRW


def _nms_scan_body(sboxes_ref, ord_ref, sc_ref, pad_ref,
                   out_ref, obox_ref, oscr_ref, kept_ref, kmeta_ref):
    lane_i = lax.broadcasted_iota(jnp.int32, (1, BLK), 1)
    sub_b = lax.broadcasted_iota(jnp.int32, (BLK, 1), 0).astype(jnp.float32)
    row_c = lax.broadcasted_iota(jnp.int32, (KCH, 1), 0)
    eye = (lax.broadcasted_iota(jnp.int32, (BLK, BLK), 0)
           == lax.broadcasted_iota(jnp.int32, (BLK, BLK), 1)).astype(jnp.float32)
    lt = (lax.broadcasted_iota(jnp.int32, (BLK, BLK), 0)
          <= lax.broadcasted_iota(jnp.int32, (BLK, BLK), 1)).astype(jnp.float32)

    def exact_mm(a, b, dims):
        return lax.dot_general(a, b, (dims, ((), ())),
                               precision=lax.Precision.HIGHEST,
                               preferred_element_type=jnp.float32)

    def body(state):
        b, cnt = state
        blk = sboxes_ref[pl.ds(b, 1)].reshape(4, BLK)
        ord_row = ord_ref[pl.ds(b, 1)].reshape(1, BLK)
        sc_row = sc_ref[pl.ds(b, 1)].reshape(1, BLK)
        cx1, cy1 = blk[0:1, :], blk[1:2, :]
        cx2, cy2 = blk[2:3, :], blk[3:4, :]
        area_c = (cx2 - cx1) * (cy2 - cy1)

        def kept_chunk(c):
            lo = c * KCH
            kx1 = kept_ref[0, lo:lo + KCH, :]
            ky1 = kept_ref[1, lo:lo + KCH, :]
            kx2 = kept_ref[2, lo:lo + KCH, :]
            ky2 = kept_ref[3, lo:lo + KCH, :]
            area_k = kept_ref[4, lo:lo + KCH, :]
            ix1 = jnp.maximum(kx1, cx1)
            iy1 = jnp.maximum(ky1, cy1)
            ix2 = jnp.minimum(kx2, cx2)
            iy2 = jnp.minimum(ky2, cy2)
            inter = jnp.maximum(ix2 - ix1, 0.0) * jnp.maximum(iy2 - iy1, 0.0)
            iou = inter / ((area_k + area_c) - inter + 1e-9)
            supp = (iou > IOU_THRESH) & ((row_c + lo) < cnt)
            return jnp.max(supp.astype(jnp.float32), axis=0, keepdims=True)

        supp_any = kept_chunk(0)
        for c in range(1, NKCH):
            m = lax.cond(cnt > c * KCH, lambda c=c: kept_chunk(c),
                         lambda: jnp.zeros((1, BLK), jnp.float32))
            supp_any = jnp.maximum(supp_any, m)
        gidx = b * BLK + lane_i
        alive = (supp_any < 0.5) & (gidx < N)

        candT = exact_mm(eye, blk, ((1,), (1,)))
        ax1, ay1 = candT[:, 0:1], candT[:, 1:2]
        ax2, ay2 = candT[:, 2:3], candT[:, 3:4]
        area_a = (ax2 - ax1) * (ay2 - ay1)
        jx1 = jnp.maximum(ax1, cx1)
        jy1 = jnp.maximum(ay1, cy1)
        jx2 = jnp.minimum(ax2, cx2)
        jy2 = jnp.minimum(ay2, cy2)
        jinter = jnp.maximum(jx2 - jx1, 0.0) * jnp.maximum(jy2 - jy1, 0.0)
        jiou = jinter / ((area_a + area_c) - jinter + 1e-9)
        sub_i2 = lax.broadcasted_iota(jnp.int32, (BLK, BLK), 0)
        lane_i2 = lax.broadcasted_iota(jnp.int32, (BLK, BLK), 1)
        sup_f = ((jiou > IOU_THRESH) & (lane_i2 > sub_i2)).astype(jnp.float32)

        base = alive

        def fix_body(st):
            a_f, _ = st
            killed = lax.dot_general(a_f, sup_f,
                                     (((1,), (0,)), ((), ())),
                                     preferred_element_type=jnp.float32)
            a_new = (base & (killed < 0.5)).astype(jnp.float32)
            return a_new, jnp.any(a_new != a_f)

        alive_f0 = base.astype(jnp.float32)
        alive_fix, _ = lax.while_loop(lambda st: st[1], fix_body,
                                      (alive_f0, jnp.bool_(True)))
        alive = alive_fix > 0.5

        alive_f = alive.astype(jnp.float32)
        cum = exact_mm(alive_f, lt, ((1,), (0,)))
        n_alive = jnp.sum(alive_f).astype(jnp.int32)
        onehot = ((sub_b == (cum - 1.0)) & alive).astype(jnp.float32)
        geom = jnp.concatenate([candT, area_a], axis=1)
        geom_new = exact_mm(onehot, geom, ((1,), (0,)))
        for c in range(5):
            kept_ref[c, pl.ds(cnt, BLK), :] = jnp.broadcast_to(
                geom_new[:, c:c + 1], (BLK, BLK))
        rows = jnp.concatenate([ord_row, sc_row], axis=0)
        rows_new = exact_mm(onehot, rows, ((1,), (1,)))
        meta = jnp.concatenate(
            [geom_new[:, 0:4], rows_new,
             jnp.zeros((BLK, 2), jnp.float32)], axis=1)
        kmeta_ref[pl.ds(cnt, BLK), :] = meta
        return b + 1, cnt + n_alive

    def cond(state):
        b, cnt = state
        return (b < NB) & (cnt < MAX_OUT)

    _, cnt_fin = lax.while_loop(cond, body, (jnp.int32(0), jnp.int32(0)))

    row_g = lax.broadcasted_iota(jnp.int32, (GPAD, 1), 0)
    sel = kmeta_ref[0:GPAD, 4:5]
    out_ref[:, :] = jnp.where((row_g < cnt_fin) & (row_g < MAX_OUT),
                              sel, 0.0).astype(jnp.int32)
    row_o = lax.broadcasted_iota(jnp.int32, (MAX_OUT, 1), 0)
    live = row_o < cnt_fin
    obox_ref[:, :] = jnp.where(live, kmeta_ref[0:MAX_OUT, 0:4],
                               pad_ref[0:1, 0:4])
    oscr_ref[:, :] = jnp.where(live, kmeta_ref[0:MAX_OUT, 5:6],
                               pad_ref[0:1, 4:5])


def _nms_scan(sboxes_blk, ord_blk, sc_blk, pad_row):
    return pl.pallas_call(
        _nms_scan_body,
        out_shape=(jax.ShapeDtypeStruct((GPAD, 1), jnp.int32),
                   jax.ShapeDtypeStruct((MAX_OUT, 4), jnp.float32),
                   jax.ShapeDtypeStruct((MAX_OUT, 1), jnp.float32)),
        scratch_shapes=[
            pltpu.VMEM((5, KBUF, BLK), jnp.float32),
            pltpu.VMEM((KBUF, 8), jnp.float32),
        ],
    )(sboxes_blk, ord_blk, sc_blk, pad_row)


def _sc_layout_body(bflat_hbm, ordp_hbm, negsc_hbm,
                    out_hbm, oord_hbm, osc_hbm,
                    idx_v, il_v, tbuf_v, nsc_v, obuf_v, sbuf_v, sem):
    wid = lax.axis_index("s") * _NC + lax.axis_index("c")
    base = wid * _BPW * BLK
    pltpu.sync_copy(ordp_hbm.at[pl.ds(base, _BPW * BLK)], idx_v)
    pltpu.sync_copy(negsc_hbm.at[pl.ds(base, _BPW * BLK)], nsc_v)
    for gg in range(_BPW):
        for c in range(4):
            for h in range(BLK // 16):
                part = idx_v[pl.ds(gg * BLK + h * 16, 16)] * 4 + c
                il_v[gg, c, pl.ds(h * 16, 16)] = part
    copies = [pltpu.async_copy(bflat_hbm.at[il_v.at[gg, c]],
                               tbuf_v.at[gg, c], sem)
              for gg in range(_BPW) for c in range(4)]
    for gg in range(_BPW):
        for h in range(BLK // 16):
            chunk = idx_v[pl.ds(gg * BLK + h * 16, 16)]
            obuf_v[gg, 0, pl.ds(h * 16, 16)] = chunk.astype(jnp.float32)
            sbuf_v[gg, 0, pl.ds(h * 16, 16)] = -nsc_v[pl.ds(gg * BLK + h * 16, 16)]
    pltpu.sync_copy(obuf_v, oord_hbm.at[pl.ds(wid * _BPW, _BPW)])
    pltpu.sync_copy(sbuf_v, osc_hbm.at[pl.ds(wid * _BPW, _BPW)])
    for cp in copies:
        cp.wait()
    pltpu.sync_copy(tbuf_v, out_hbm.at[pl.ds(wid * _BPW, _BPW)])


@functools.lru_cache(maxsize=1)
def _sc_layout():
    return pl.kernel(
        _sc_layout_body,
        out_type=(jax.ShapeDtypeStruct((NBP, 4, BLK), jnp.float32),
                  jax.ShapeDtypeStruct((NBP, 1, BLK), jnp.float32),
                  jax.ShapeDtypeStruct((NBP, 1, BLK), jnp.float32)),
        mesh=plsc.VectorSubcoreMesh(core_axis_name="c", subcore_axis_name="s"),
        scratch_types=[
            pltpu.VMEM((_BPW * BLK,), jnp.int32),
            pltpu.VMEM((_BPW, 4, BLK), jnp.int32),
            pltpu.VMEM((_BPW, 4, BLK), jnp.float32),
            pltpu.VMEM((_BPW * BLK,), jnp.float32),
            pltpu.VMEM((_BPW, 1, BLK), jnp.float32),
            pltpu.VMEM((_BPW, 1, BLK), jnp.float32),
            pltpu.SemaphoreType.DMA,
        ],
        compiler_params=pltpu.CompilerParams(use_tc_tiling_on_sc=False),
    )


def _sc_gather_body(feats_hbm, sel_hbm, out_hbm, idx_v, buf, sem):
    wid = lax.axis_index("s") * _NC + lax.axis_index("c")
    base = wid * _GRW

    @pl.when(wid < _GW)
    def _():
        pltpu.sync_copy(sel_hbm.at[pl.ds(base, _GRW)], idx_v)
        pltpu.async_copy(feats_hbm.at[idx_v], buf, sem).wait()
        pltpu.sync_copy(buf, out_hbm.at[pl.ds(base, _GRW)])


@functools.lru_cache(maxsize=1)
def _sc_gather():
    return pl.kernel(
        _sc_gather_body,
        out_type=jax.ShapeDtypeStruct((MAX_OUT, C), jnp.float32),
        mesh=plsc.VectorSubcoreMesh(core_axis_name="c", subcore_axis_name="s"),
        scratch_types=[
            pltpu.VMEM((_GRW,), jnp.int32),
            pltpu.VMEM((_GRW, C), jnp.float32),
            pltpu.SemaphoreType.DMA,
        ],
        compiler_params=pltpu.CompilerParams(use_tc_tiling_on_sc=False),
    )


def kernel(boxes, scores, feats):
    iota = lax.iota(jnp.int32, N)
    neg_sorted, order = lax.sort((-scores, iota), num_keys=1)
    order_pad = jnp.pad(order, (0, NBP * BLK - N))
    negsc_pad = jnp.pad(neg_sorted, (0, NBP * BLK - N))
    sboxes_blk, ord_blk, sc_blk = _sc_layout()(boxes.reshape(-1), order_pad,
                                               negsc_pad)
    pad_row = jnp.concatenate(
        [boxes[0:1, :], scores[0:1, None], jnp.zeros((1, 3), jnp.float32)],
        axis=1)

    sel, kept_boxes, kept_scores = _nms_scan(sboxes_blk, ord_blk, sc_blk,
                                             pad_row)

    feats_g = _sc_gather()(feats, sel.reshape(-1))

    return kept_boxes, kept_scores.reshape(-1), feats_g
